# baseline (speedup 1.0000x reference)
import functools
import math

import jax
import jax.numpy as jnp
from jax import lax
from jax.experimental import pallas as pl
from jax.experimental.pallas import tpu as pltpu
from jax.experimental.pallas import tpu_sc as plsc

N = 4096
D = 128
K = 32
SCALE = 0.5
DESTNUM = 10
THRESHOLD = 1.0

RB = 256
NB = N // RB
SIZES = (256, 256, 512, 1024, 2048)
NW = 32
RPW = N // NW

_INTERPRET = False


def _stage_base(s):
    return N - sum(SIZES[: s + 1])


def _k1_body(x_blk, xT, sq_col, sq_row, offsum_ref, diag_ref):
    i = pl.program_id(0)
    g = jnp.dot(x_blk[...], xT[...], preferred_element_type=jnp.float32)
    d2 = jnp.maximum(sq_col[...] + sq_row[...] - 2.0 * g, 0.0)
    col = lax.broadcasted_iota(jnp.int32, (RB, N), 1)
    row = lax.broadcasted_iota(jnp.int32, (RB, N), 0) + i * RB
    ondiag = col == row
    e = jnp.exp(-2.0 * d2)
    offsum = jnp.sum(jnp.where(ondiag, 0.0, e), axis=1)
    diag = jnp.sum(jnp.where(ondiag, d2, 0.0), axis=1)
    offsum_ref[...] = offsum.reshape(1, 1, RB)
    diag_ref[...] = diag.reshape(1, 1, RB)


def _k1(x, sq):
    f = pl.pallas_call(
        _k1_body,
        grid=(NB,),
        in_specs=[
            pl.BlockSpec((RB, D), lambda i: (i, 0)),
            pl.BlockSpec((D, N), lambda i: (0, 0)),
            pl.BlockSpec((RB, 1), lambda i: (i, 0)),
            pl.BlockSpec((1, N), lambda i: (0, 0)),
        ],
        out_specs=[
            pl.BlockSpec((1, 1, RB), lambda i: (i, 0, 0)),
            pl.BlockSpec((1, 1, RB), lambda i: (i, 0, 0)),
        ],
        out_shape=[
            jax.ShapeDtypeStruct((NB, 1, RB), jnp.float32),
            jax.ShapeDtypeStruct((NB, 1, RB), jnp.float32),
        ],
        interpret=_INTERPRET,
    )
    offsum, diag = f(x, x.T, sq.reshape(N, 1), sq.reshape(1, N))
    return offsum.reshape(N), diag.reshape(N)


DAUG = 2 * D


def _k2_body(r_col, r_row, x_blk, sq_col, rank_ref, xaug_ref):
    i = pl.program_id(0)
    mx = jnp.max(r_row[...])
    di = r_col[...] / mx
    dj = r_row[...] / mx
    lt = (dj < di).astype(jnp.int32)
    col = lax.broadcasted_iota(jnp.int32, (RB, N), 1)
    row = lax.broadcasted_iota(jnp.int32, (RB, N), 0) + i * RB
    eqlow = ((dj == di) & (col < row)).astype(jnp.int32)
    rank = jnp.sum(lt + eqlow, axis=1)
    rank_ref[...] = rank.reshape(1, 1, RB)
    lane = lax.broadcasted_iota(jnp.int32, (RB, DAUG), 1)
    xaug_ref[:, : D] = x_blk[...]
    tail = jnp.where(lane[:, D: 2 * D] == D, sq_col[...],
                     jnp.where(lane[:, D: 2 * D] == D + 1, di, 0.0))
    xaug_ref[:, D: 2 * D] = tail
    return


def _k2(res, x, sq):
    f = pl.pallas_call(
        _k2_body,
        grid=(NB,),
        in_specs=[
            pl.BlockSpec((RB, 1), lambda i: (i, 0)),
            pl.BlockSpec((1, N), lambda i: (0, 0)),
            pl.BlockSpec((RB, D), lambda i: (i, 0)),
            pl.BlockSpec((RB, 1), lambda i: (i, 0)),
        ],
        out_specs=[
            pl.BlockSpec((1, 1, RB), lambda i: (i, 0, 0)),
            pl.BlockSpec((RB, DAUG), lambda i: (i, 0)),
        ],
        out_shape=[
            jax.ShapeDtypeStruct((NB, 1, RB), jnp.int32),
            jax.ShapeDtypeStruct((N, DAUG), jnp.float32),
        ],
        interpret=_INTERPRET,
    )
    rank, xaug = f(res.reshape(N, 1), res.reshape(1, N), x,
                   sq.reshape(N, 1))
    return rank.reshape(N), xaug


def _k3_body(xaug_hbm, rank_hbm, out_hbm, idx_v, rows_v, sem):
    wid = lax.axis_index("s") * 2 + lax.axis_index("c")
    base = wid * RPW
    pltpu.sync_copy(rank_hbm.at[wid], idx_v)
    pltpu.sync_copy(xaug_hbm.at[pl.ds(base, RPW)], rows_v)
    pltpu.async_copy(rows_v, out_hbm.at[idx_v], sem).wait()


def _k3(xaug, rank):
    mesh = plsc.VectorSubcoreMesh(core_axis_name="c", subcore_axis_name="s", num_cores=2, num_subcores=16)
    f = pl.kernel(
        _k3_body,
        out_type=jax.ShapeDtypeStruct((N, DAUG), jnp.float32),
        mesh=mesh,
        scratch_types=[
            pltpu.VMEM((RPW,), jnp.int32),
            pltpu.VMEM((RPW, DAUG), jnp.float32),
            pltpu.SemaphoreType.DMA,
        ],
        compiler_params=pltpu.CompilerParams(needs_layout_passes=False),
        interpret=_INTERPRET,
    )
    return f(xaug, rank.reshape(NW, RPW))


BIGF = float("inf")
BIGI = 1 << 30


def _k4_body(xs_blk, xsT, sqs_col, sqs_row, rips_ref, d2_scr):
    g = jnp.dot(xs_blk[...], xsT[...], preferred_element_type=jnp.float32)
    d2_scr[...] = jnp.maximum(sqs_col[...] + sqs_row[...] - 2.0 * g, 0.0)
    col = lax.broadcasted_iota(jnp.int32, (RB, N), 1)
    lane = lax.broadcasted_iota(jnp.int32, (RB, K), 1)

    def round_body(r, _):
        v = d2_scr[...]
        m = jnp.min(v, axis=1, keepdims=True)
        at_min = v == m
        midx = jnp.min(jnp.where(at_min, col, jnp.int32(BIGI)),
                       axis=1, keepdims=True)
        rips_ref[...] = jnp.where(lane == r, midx, rips_ref[...])
        d2_scr[...] = jnp.where(col == midx, jnp.float32(BIGF), v)
        return 0

    lax.fori_loop(0, K, round_body, 0)


def _k4(s, xs, xsT, sqs_col, sqs_row):
    size = SIZES[s]
    off = _stage_base(s) // RB
    f = pl.pallas_call(
        _k4_body,
        grid=(size // RB,),
        in_specs=[
            pl.BlockSpec((RB, D), lambda i, off=off: (i + off, 0)),
            pl.BlockSpec((D, N), lambda i: (0, 0)),
            pl.BlockSpec((RB, 1), lambda i, off=off: (i + off, 0)),
            pl.BlockSpec((1, N), lambda i: (0, 0)),
        ],
        out_specs=pl.BlockSpec((RB, K), lambda i: (i, 0)),
        out_shape=jax.ShapeDtypeStruct((size, K), jnp.int32),
        scratch_shapes=[
            pltpu.VMEM((RB, N), jnp.float32),
        ],
        interpret=_INTERPRET,
    )
    return f(xs, xsT, sqs_col, sqs_row)


NEG = float("-inf")


def _lane0():
    return lax.iota(jnp.int32, 16) == 0


def _splat(x):
    return jnp.full((16,), x, dtype=jnp.int32)


def _k5_rows(base, size, ch, guard_top, rips_hbm, pte, ms, kde_v, chunk):
    nch = size // ch

    def chunk_body(c2, _):
        cid = nch - 1 - c2
        pltpu.sync_copy(rips_hbm.at[pl.ds(cid * ch, ch)], chunk)

        def row_body(s2, _):
            rr = ch - 1 - s2
            i = base + cid * ch + rr

            def process():
                nb0 = chunk[rr, pl.ds(0, 16)]
                nb1 = chunk[rr, pl.ds(16, 16)]
                g = lax.reduce_max(jnp.maximum(nb0, nb1), (0,))

                @pl.when(g > i)
                def _():
                    iv = _splat(i)
                    m0 = nb0 > iv
                    m1 = nb1 > iv

                    def find_cond(carry):
                        _, _, ch_ = carry
                        return ch_ > 0

                    def find_step(carry):
                        r0, r1, _ = carry
                        t0 = plsc.load_gather(pte, [r0])
                        t1 = plsc.load_gather(pte, [r1])
                        ch_ = lax.reduce_max(
                            jnp.where((t0 != r0) | (t1 != r1),
                                      1, 0).astype(jnp.int32), (0,))
                        return (t0, t1, ch_)

                    r0, r1, _ = lax.while_loop(
                        find_cond, find_step,
                        (nb0, nb1, jnp.int32(1)))
                    plsc.store_scatter(pte, [nb0], r0)
                    plsc.store_scatter(pte, [nb1], r1)
                    gv = jnp.full((16,), g, jnp.int32)
                    e_up = lax.reduce_max(
                        jnp.maximum(jnp.where(nb0 == gv, r0, -1),
                                    jnp.where(nb1 == gv, r1, -1)), (0,))
                    ev = jnp.full((16,), e_up, jnp.int32)
                    plsc.store_scatter(pte, [iv], ev, mask=_lane0())
                    kde_i = plsc.load_gather(kde_v, [iv])
                    for nb, m, r in ((nb0, m0, r0), (nb1, m1, r1)):
                        diff = m & (r != ev)
                        kde_r = plsc.load_gather(kde_v, [r])
                        merge = diff & ((kde_r - kde_i)
                                        < jnp.float32(THRESHOLD))
                        ms_old = plsc.load_gather(ms, [r])
                        plsc.store_scatter(ms, [r],
                                           jnp.maximum(ms_old, iv),
                                           mask=diff)
                        plsc.store_scatter(pte, [r], ev, mask=merge)

            if guard_top:
                pl.when(i <= N - 2)(process)
            else:
                process()
            return 0

        lax.fori_loop(0, ch, row_body, 0)
        return 0

    lax.fori_loop(0, nch, chunk_body, 0)


def _k5_epilogue(pte, ms, kde_v, pers_b, sec_b, outv,
                 misc_hbm, topa_hbm, topb_hbm, topv_hbm):
    base16 = lax.iota(jnp.int32, 16)
    nv = _splat(N)

    def pers_body(p, carry):
        s_all, n_app = carry
        o = p * 16
        a = kde_v[pl.ds(o, 16)]
        msv = ms[pl.ds(o, 16)]
        av = msv >= 0
        msw = jnp.where(msv < 0, msv + nv, msv)
        b = plsc.load_gather(kde_v, [msw])
        d = a - b
        pers_b[pl.ds(o, 16)] = jnp.where(av, d, NEG)
        sec_b[pl.ds(o, 16)] = b
        s_all = s_all + lax.reduce_sum(jnp.where(av, d, 0.0), (0,))
        n_app = n_app + lax.reduce_sum(
            jnp.where(av, 1, 0).astype(jnp.int32), (0,))
        return (s_all, n_app)

    s_all, n_app = lax.fori_loop(0, N // 16, pers_body,
                                 (jnp.float32(0.0), jnp.int32(0)))

    def sel_body(t, _):
        def mx_body(p, acc):
            return jnp.maximum(acc, pers_b[pl.ds(p * 16, 16)])

        mv = lax.fori_loop(0, N // 16, mx_body,
                           jnp.full((16,), NEG, jnp.float32))
        mval = lax.reduce_max(mv, (0,))
        mvv = jnp.full((16,), mval, jnp.float32)

        def ix_body(p, acc):
            o = p * 16
            pv = pers_b[pl.ds(o, 16)]
            gidx = base16 + o
            return jnp.maximum(acc, jnp.where(pv == mvv, gidx, -1))

        iv = lax.fori_loop(0, N // 16, ix_body,
                           jnp.full((16,), -1, jnp.int32))
        midx = lax.reduce_max(iv, (0,))
        mi = jnp.full((16,), midx, jnp.int32)
        a_m = plsc.load_gather(kde_v, [mi])
        b_m = plsc.load_gather(sec_b, [mi])
        ms_m = plsc.load_gather(ms, [mi])
        v_m = jnp.where(ms_m >= 0, jnp.float32(1.0), jnp.float32(0.0))
        tv = _splat(t)
        plsc.store_scatter(outv, [tv], a_m, mask=_lane0())
        plsc.store_scatter(outv, [tv + _splat(16)], b_m, mask=_lane0())
        plsc.store_scatter(outv, [tv + _splat(32)], v_m, mask=_lane0())
        plsc.store_scatter(pers_b, [mi],
                           jnp.full((16,), NEG, jnp.float32),
                           mask=_lane0())
        return 0

    lax.fori_loop(0, DESTNUM, sel_body, 0)

    misc = jnp.where(base16 == 0, jnp.full((16,), s_all, jnp.float32),
                     jnp.where(base16 == 1,
                               jnp.full((16,), jnp.float32(n_app)),
                               jnp.zeros((16,), jnp.float32)))
    outv[pl.ds(48, 16)] = misc
    pltpu.sync_copy(outv.at[pl.ds(0, 16)], topa_hbm)
    pltpu.sync_copy(outv.at[pl.ds(16, 16)], topb_hbm)
    pltpu.sync_copy(outv.at[pl.ds(32, 16)], topv_hbm)
    pltpu.sync_copy(outv.at[pl.ds(48, 16)], misc_hbm)


def _k5_body_mid(s, ch, rips_hbm, kde_hbm, pte_in, ms_in, pte_hbm, ms_hbm,
                 pte, ms, kde_v, chunk):
    wid = lax.axis_index("s") * 2 + lax.axis_index("c")

    @pl.when(wid == 0)
    def _():
        pltpu.sync_copy(kde_hbm, kde_v)
        pltpu.sync_copy(pte_in, pte)
        pltpu.sync_copy(ms_in, ms)
        _k5_rows(_stage_base(s), SIZES[s], ch, s == 0,
                 rips_hbm, pte, ms, kde_v, chunk)
        pltpu.sync_copy(pte, pte_hbm)
        pltpu.sync_copy(ms, ms_hbm)


def _k5_body_last(s, ch, rips_hbm, kde_hbm, pte_in, ms_in,
                  misc_hbm, topa_hbm, topb_hbm, topv_hbm,
                  pte, ms, kde_v, pers_b, sec_b, chunk, outv):
    wid = lax.axis_index("s") * 2 + lax.axis_index("c")

    @pl.when(wid == 0)
    def _():
        pltpu.sync_copy(kde_hbm, kde_v)
        pltpu.sync_copy(pte_in, pte)
        pltpu.sync_copy(ms_in, ms)
        _k5_rows(0, SIZES[s], ch, False, rips_hbm, pte, ms, kde_v, chunk)
        _k5_epilogue(pte, ms, kde_v, pers_b, sec_b, outv,
                     misc_hbm, topa_hbm, topb_hbm, topv_hbm)


def _sc_mesh():
    return plsc.VectorSubcoreMesh(core_axis_name="c", subcore_axis_name="s",
                                  num_cores=2, num_subcores=16)


def _k5(s, rips_s, kde, state):
    size = SIZES[s]
    ch = 512 if size % 512 == 0 else 256
    base_scratch = [
        pltpu.VMEM((N,), jnp.int32),
        pltpu.VMEM((N,), jnp.int32),
        pltpu.VMEM((N,), jnp.float32),
    ]
    if s < len(SIZES) - 1:
        f = pl.kernel(
            functools.partial(_k5_body_mid, s, ch),
            out_type=[
                jax.ShapeDtypeStruct((N,), jnp.int32),
                jax.ShapeDtypeStruct((N,), jnp.int32),
            ],
            mesh=_sc_mesh(),
            scratch_types=base_scratch + [
                pltpu.VMEM((ch, K), jnp.int32),
            ],
            compiler_params=pltpu.CompilerParams(needs_layout_passes=False),
            interpret=_INTERPRET,
        )
        return f(rips_s, kde, state[0], state[1])
    f = pl.kernel(
        functools.partial(_k5_body_last, s, ch),
        out_type=[
            jax.ShapeDtypeStruct((16,), jnp.float32),
            jax.ShapeDtypeStruct((16,), jnp.float32),
            jax.ShapeDtypeStruct((16,), jnp.float32),
            jax.ShapeDtypeStruct((16,), jnp.float32),
        ],
        mesh=_sc_mesh(),
        scratch_types=base_scratch + [
            pltpu.VMEM((N,), jnp.float32),
            pltpu.VMEM((N,), jnp.float32),
            pltpu.VMEM((ch, K), jnp.int32),
            pltpu.VMEM((64,), jnp.float32),
        ],
        compiler_params=pltpu.CompilerParams(needs_layout_passes=False),
        interpret=_INTERPRET,
    )
    return f(rips_s, kde, state[0], state[1])


def kernel(x):
    x = jnp.asarray(x, jnp.float32)
    sq = jnp.sum(x * x, axis=1)
    offsum, diag = _k1(x, sq)
    res = (offsum + jnp.exp(-jnp.maximum(diag, 0.0) / SCALE)) / (K * SCALE)
    rank, xaug = _k2(res, x, sq)
    xaug_s = _k3(xaug, rank)
    xs = xaug_s[:, :D]
    sqs = xaug_s[:, D]
    kde = xaug_s[:, D + 1]
    xsT = xs.T
    sqs_col = sqs.reshape(N, 1)
    sqs_row = sqs.reshape(1, N)
    state = (jnp.arange(N, dtype=jnp.int32), jnp.full((N,), -1, jnp.int32))
    for s in range(len(SIZES)):
        rips_s = _k4(s, xs, xsT, sqs_col, sqs_row)
        if s < len(SIZES) - 1:
            state = _k5(s, rips_s, kde, state)
        else:
            misc, topa, topb, topv = _k5(s, rips_s, kde, state)
    s_all, n_app = misc[0], misc[1]
    valid = topv[:DESTNUM] > 0.5
    a = topa[:DESTNUM]
    b = topb[:DESTNUM]
    s_top = jnp.sum(jnp.where(valid, a - b, 0.0))
    weak = (s_all - s_top) / math.sqrt(2.0)
    dest_a, dest_b = topa[0], topb[0]
    dists = jnp.sqrt((a - dest_a) ** 2 + (b - dest_b) ** 2)
    strong = jnp.sum(jnp.where(valid, dists, 0.0))
    return jnp.where(n_app > 0.5, weak + strong, jnp.float32(0.0))


# speedup vs baseline: 1.0510x; 1.0510x over previous
import functools
import math

import jax
import jax.numpy as jnp
from jax import lax
from jax.experimental import pallas as pl
from jax.experimental.pallas import tpu as pltpu
from jax.experimental.pallas import tpu_sc as plsc

N = 4096
D = 128
K = 32
SCALE = 0.5
DESTNUM = 10
THRESHOLD = 1.0

RB = 256
NB = N // RB
SIZES = (256, 512, 1024, 1024, 1280)
NW = 32
RPW = N // NW

_INTERPRET = False


def _stage_base(s):
    return N - sum(SIZES[: s + 1])


def _k1_body(x_blk, xT, sq_col, sq_row, offsum_ref, diag_ref):
    i = pl.program_id(0)
    g = jnp.dot(x_blk[...], xT[...], preferred_element_type=jnp.float32)
    d2 = jnp.maximum(sq_col[...] + sq_row[...] - 2.0 * g, 0.0)
    col = lax.broadcasted_iota(jnp.int32, (RB, N), 1)
    row = lax.broadcasted_iota(jnp.int32, (RB, N), 0) + i * RB
    ondiag = col == row
    e = jnp.exp(-2.0 * d2)
    offsum = jnp.sum(jnp.where(ondiag, 0.0, e), axis=1)
    diag = jnp.sum(jnp.where(ondiag, d2, 0.0), axis=1)
    offsum_ref[...] = offsum.reshape(1, 1, RB)
    diag_ref[...] = diag.reshape(1, 1, RB)


def _k1(x, sq):
    f = pl.pallas_call(
        _k1_body,
        grid=(NB,),
        in_specs=[
            pl.BlockSpec((RB, D), lambda i: (i, 0)),
            pl.BlockSpec((D, N), lambda i: (0, 0)),
            pl.BlockSpec((RB, 1), lambda i: (i, 0)),
            pl.BlockSpec((1, N), lambda i: (0, 0)),
        ],
        out_specs=[
            pl.BlockSpec((1, 1, RB), lambda i: (i, 0, 0)),
            pl.BlockSpec((1, 1, RB), lambda i: (i, 0, 0)),
        ],
        out_shape=[
            jax.ShapeDtypeStruct((NB, 1, RB), jnp.float32),
            jax.ShapeDtypeStruct((NB, 1, RB), jnp.float32),
        ],
        interpret=_INTERPRET,
    )
    offsum, diag = f(x, x.T, sq.reshape(N, 1), sq.reshape(1, N))
    return offsum.reshape(N), diag.reshape(N)


DAUG = 2 * D


def _k2_body(r_col, r_row, x_blk, sq_col, rank_ref, xaug_ref):
    i = pl.program_id(0)
    mx = jnp.max(r_row[...])
    di = r_col[...] / mx
    dj = r_row[...] / mx
    lt = (dj < di).astype(jnp.int32)
    col = lax.broadcasted_iota(jnp.int32, (RB, N), 1)
    row = lax.broadcasted_iota(jnp.int32, (RB, N), 0) + i * RB
    eqlow = ((dj == di) & (col < row)).astype(jnp.int32)
    rank = jnp.sum(lt + eqlow, axis=1)
    rank_ref[...] = rank.reshape(1, 1, RB)
    lane = lax.broadcasted_iota(jnp.int32, (RB, DAUG), 1)
    xaug_ref[:, : D] = x_blk[...]
    tail = jnp.where(lane[:, D: 2 * D] == D, sq_col[...],
                     jnp.where(lane[:, D: 2 * D] == D + 1, di, 0.0))
    xaug_ref[:, D: 2 * D] = tail
    return


def _k2(res, x, sq):
    f = pl.pallas_call(
        _k2_body,
        grid=(NB,),
        in_specs=[
            pl.BlockSpec((RB, 1), lambda i: (i, 0)),
            pl.BlockSpec((1, N), lambda i: (0, 0)),
            pl.BlockSpec((RB, D), lambda i: (i, 0)),
            pl.BlockSpec((RB, 1), lambda i: (i, 0)),
        ],
        out_specs=[
            pl.BlockSpec((1, 1, RB), lambda i: (i, 0, 0)),
            pl.BlockSpec((RB, DAUG), lambda i: (i, 0)),
        ],
        out_shape=[
            jax.ShapeDtypeStruct((NB, 1, RB), jnp.int32),
            jax.ShapeDtypeStruct((N, DAUG), jnp.float32),
        ],
        interpret=_INTERPRET,
    )
    rank, xaug = f(res.reshape(N, 1), res.reshape(1, N), x,
                   sq.reshape(N, 1))
    return rank.reshape(N), xaug


def _k3_body(xaug_hbm, rank_hbm, out_hbm, idx_v, rows_v, sem):
    wid = lax.axis_index("s") * 2 + lax.axis_index("c")
    base = wid * RPW
    pltpu.sync_copy(rank_hbm.at[wid], idx_v)
    pltpu.sync_copy(xaug_hbm.at[pl.ds(base, RPW)], rows_v)
    pltpu.async_copy(rows_v, out_hbm.at[idx_v], sem).wait()


def _k3(xaug, rank):
    mesh = plsc.VectorSubcoreMesh(core_axis_name="c", subcore_axis_name="s", num_cores=2, num_subcores=16)
    f = pl.kernel(
        _k3_body,
        out_type=jax.ShapeDtypeStruct((N, DAUG), jnp.float32),
        mesh=mesh,
        scratch_types=[
            pltpu.VMEM((RPW,), jnp.int32),
            pltpu.VMEM((RPW, DAUG), jnp.float32),
            pltpu.SemaphoreType.DMA,
        ],
        compiler_params=pltpu.CompilerParams(needs_layout_passes=False),
        interpret=_INTERPRET,
    )
    return f(xaug, rank.reshape(NW, RPW))


BIGF = float("inf")
BIGI = 1 << 30


def _k4_body(xs_blk, xsT, sqs_col, sqs_row, rips_ref, d2_scr):
    g = jnp.dot(xs_blk[...], xsT[...], preferred_element_type=jnp.float32)
    d2_scr[...] = jnp.maximum(sqs_col[...] + sqs_row[...] - 2.0 * g, 0.0)
    col = lax.broadcasted_iota(jnp.int32, (RB, N), 1)
    lane = lax.broadcasted_iota(jnp.int32, (RB, K), 1)

    def round_body(r, _):
        v = d2_scr[...]
        m = jnp.min(v, axis=1, keepdims=True)
        at_min = v == m
        midx = jnp.min(jnp.where(at_min, col, jnp.int32(BIGI)),
                       axis=1, keepdims=True)
        rips_ref[...] = jnp.where(lane == r, midx, rips_ref[...])
        d2_scr[...] = jnp.where(col == midx, jnp.float32(BIGF), v)
        return 0

    lax.fori_loop(0, K, round_body, 0)


def _k4(s, xs, xsT, sqs_col, sqs_row):
    size = SIZES[s]
    off = _stage_base(s) // RB
    f = pl.pallas_call(
        _k4_body,
        grid=(size // RB,),
        in_specs=[
            pl.BlockSpec((RB, D), lambda i, off=off: (i + off, 0)),
            pl.BlockSpec((D, N), lambda i: (0, 0)),
            pl.BlockSpec((RB, 1), lambda i, off=off: (i + off, 0)),
            pl.BlockSpec((1, N), lambda i: (0, 0)),
        ],
        out_specs=pl.BlockSpec((RB, K), lambda i: (i, 0)),
        out_shape=jax.ShapeDtypeStruct((size, K), jnp.int32),
        scratch_shapes=[
            pltpu.VMEM((RB, N), jnp.float32),
        ],
        interpret=_INTERPRET,
    )
    return f(xs, xsT, sqs_col, sqs_row)


NEG = float("-inf")


def _lane0():
    return lax.iota(jnp.int32, 16) == 0


def _splat(x):
    return jnp.full((16,), x, dtype=jnp.int32)


def _k5_rows(base, size, ch, guard_top, rips_hbm, pte, ms, kde_v, chunk):
    nch = size // ch

    def chunk_body(c2, _):
        cid = nch - 1 - c2
        pltpu.sync_copy(rips_hbm.at[pl.ds(cid * ch, ch)], chunk)

        def row_body(s2, _):
            rr = ch - 1 - s2
            i = base + cid * ch + rr

            def process():
                nb0 = chunk[rr, pl.ds(0, 16)]
                nb1 = chunk[rr, pl.ds(16, 16)]
                g = lax.reduce_max(jnp.maximum(nb0, nb1), (0,))

                @pl.when(g > i)
                def _():
                    iv = _splat(i)
                    m0 = nb0 > iv
                    m1 = nb1 > iv

                    def find_cond(carry):
                        _, _, ch_ = carry
                        return ch_ > 0

                    def find_step(carry):
                        r0, r1, _ = carry
                        t0 = plsc.load_gather(pte, [r0])
                        t1 = plsc.load_gather(pte, [r1])
                        ch_ = lax.reduce_max(
                            jnp.where((t0 != r0) | (t1 != r1),
                                      1, 0).astype(jnp.int32), (0,))
                        return (t0, t1, ch_)

                    r0, r1, _ = lax.while_loop(
                        find_cond, find_step,
                        (nb0, nb1, jnp.int32(1)))
                    plsc.store_scatter(pte, [nb0], r0)
                    plsc.store_scatter(pte, [nb1], r1)
                    gv = jnp.full((16,), g, jnp.int32)
                    e_up = lax.reduce_max(
                        jnp.maximum(jnp.where(nb0 == gv, r0, -1),
                                    jnp.where(nb1 == gv, r1, -1)), (0,))
                    ev = jnp.full((16,), e_up, jnp.int32)
                    plsc.store_scatter(pte, [iv], ev, mask=_lane0())
                    kde_i = plsc.load_gather(kde_v, [iv])
                    for nb, m, r in ((nb0, m0, r0), (nb1, m1, r1)):
                        diff = m & (r != ev)
                        kde_r = plsc.load_gather(kde_v, [r])
                        merge = diff & ((kde_r - kde_i)
                                        < jnp.float32(THRESHOLD))
                        ms_old = plsc.load_gather(ms, [r])
                        plsc.store_scatter(ms, [r],
                                           jnp.maximum(ms_old, iv),
                                           mask=diff)
                        plsc.store_scatter(pte, [r], ev, mask=merge)

            if guard_top:
                pl.when(i <= N - 2)(process)
            else:
                process()
            return 0

        lax.fori_loop(0, ch, row_body, 0)
        return 0

    lax.fori_loop(0, nch, chunk_body, 0)


def _k5_epilogue(pte, ms, kde_v, pers_b, sec_b, outv,
                 misc_hbm, topa_hbm, topb_hbm, topv_hbm):
    base16 = lax.iota(jnp.int32, 16)
    nv = _splat(N)

    def pers_body(p, carry):
        s_all, n_app = carry
        o = p * 16
        a = kde_v[pl.ds(o, 16)]
        msv = ms[pl.ds(o, 16)]
        av = msv >= 0
        msw = jnp.where(msv < 0, msv + nv, msv)
        b = plsc.load_gather(kde_v, [msw])
        d = a - b
        pers_b[pl.ds(o, 16)] = jnp.where(av, d, NEG)
        sec_b[pl.ds(o, 16)] = b
        s_all = s_all + lax.reduce_sum(jnp.where(av, d, 0.0), (0,))
        n_app = n_app + lax.reduce_sum(
            jnp.where(av, 1, 0).astype(jnp.int32), (0,))
        return (s_all, n_app)

    s_all, n_app = lax.fori_loop(0, N // 16, pers_body,
                                 (jnp.float32(0.0), jnp.int32(0)))

    def sel_body(t, _):
        def mx_body(p, acc):
            return jnp.maximum(acc, pers_b[pl.ds(p * 16, 16)])

        mv = lax.fori_loop(0, N // 16, mx_body,
                           jnp.full((16,), NEG, jnp.float32))
        mval = lax.reduce_max(mv, (0,))
        mvv = jnp.full((16,), mval, jnp.float32)

        def ix_body(p, acc):
            o = p * 16
            pv = pers_b[pl.ds(o, 16)]
            gidx = base16 + o
            return jnp.maximum(acc, jnp.where(pv == mvv, gidx, -1))

        iv = lax.fori_loop(0, N // 16, ix_body,
                           jnp.full((16,), -1, jnp.int32))
        midx = lax.reduce_max(iv, (0,))
        mi = jnp.full((16,), midx, jnp.int32)
        a_m = plsc.load_gather(kde_v, [mi])
        b_m = plsc.load_gather(sec_b, [mi])
        ms_m = plsc.load_gather(ms, [mi])
        v_m = jnp.where(ms_m >= 0, jnp.float32(1.0), jnp.float32(0.0))
        tv = _splat(t)
        plsc.store_scatter(outv, [tv], a_m, mask=_lane0())
        plsc.store_scatter(outv, [tv + _splat(16)], b_m, mask=_lane0())
        plsc.store_scatter(outv, [tv + _splat(32)], v_m, mask=_lane0())
        plsc.store_scatter(pers_b, [mi],
                           jnp.full((16,), NEG, jnp.float32),
                           mask=_lane0())
        return 0

    lax.fori_loop(0, DESTNUM, sel_body, 0)

    misc = jnp.where(base16 == 0, jnp.full((16,), s_all, jnp.float32),
                     jnp.where(base16 == 1,
                               jnp.full((16,), jnp.float32(n_app)),
                               jnp.zeros((16,), jnp.float32)))
    outv[pl.ds(48, 16)] = misc
    pltpu.sync_copy(outv.at[pl.ds(0, 16)], topa_hbm)
    pltpu.sync_copy(outv.at[pl.ds(16, 16)], topb_hbm)
    pltpu.sync_copy(outv.at[pl.ds(32, 16)], topv_hbm)
    pltpu.sync_copy(outv.at[pl.ds(48, 16)], misc_hbm)


def _k5_body_mid(s, ch, rips_hbm, kde_hbm, pte_in, ms_in, pte_hbm, ms_hbm,
                 pte, ms, kde_v, chunk):
    wid = lax.axis_index("s") * 2 + lax.axis_index("c")

    @pl.when(wid == 0)
    def _():
        pltpu.sync_copy(kde_hbm, kde_v)
        pltpu.sync_copy(pte_in, pte)
        pltpu.sync_copy(ms_in, ms)
        _k5_rows(_stage_base(s), SIZES[s], ch, s == 0,
                 rips_hbm, pte, ms, kde_v, chunk)
        pltpu.sync_copy(pte, pte_hbm)
        pltpu.sync_copy(ms, ms_hbm)


def _k5_body_last(s, ch, rips_hbm, kde_hbm, pte_in, ms_in,
                  misc_hbm, topa_hbm, topb_hbm, topv_hbm,
                  pte, ms, kde_v, pers_b, sec_b, chunk, outv):
    wid = lax.axis_index("s") * 2 + lax.axis_index("c")

    @pl.when(wid == 0)
    def _():
        pltpu.sync_copy(kde_hbm, kde_v)
        pltpu.sync_copy(pte_in, pte)
        pltpu.sync_copy(ms_in, ms)
        _k5_rows(0, SIZES[s], ch, False, rips_hbm, pte, ms, kde_v, chunk)
        _k5_epilogue(pte, ms, kde_v, pers_b, sec_b, outv,
                     misc_hbm, topa_hbm, topb_hbm, topv_hbm)


def _sc_mesh():
    return plsc.VectorSubcoreMesh(core_axis_name="c", subcore_axis_name="s",
                                  num_cores=2, num_subcores=16)


def _k5(s, rips_s, kde, state):
    size = SIZES[s]
    ch = 512 if size % 512 == 0 else 256
    base_scratch = [
        pltpu.VMEM((N,), jnp.int32),
        pltpu.VMEM((N,), jnp.int32),
        pltpu.VMEM((N,), jnp.float32),
    ]
    if s < len(SIZES) - 1:
        f = pl.kernel(
            functools.partial(_k5_body_mid, s, ch),
            out_type=[
                jax.ShapeDtypeStruct((N,), jnp.int32),
                jax.ShapeDtypeStruct((N,), jnp.int32),
            ],
            mesh=_sc_mesh(),
            scratch_types=base_scratch + [
                pltpu.VMEM((ch, K), jnp.int32),
            ],
            compiler_params=pltpu.CompilerParams(needs_layout_passes=False),
            interpret=_INTERPRET,
        )
        return f(rips_s, kde, state[0], state[1])
    f = pl.kernel(
        functools.partial(_k5_body_last, s, ch),
        out_type=[
            jax.ShapeDtypeStruct((16,), jnp.float32),
            jax.ShapeDtypeStruct((16,), jnp.float32),
            jax.ShapeDtypeStruct((16,), jnp.float32),
            jax.ShapeDtypeStruct((16,), jnp.float32),
        ],
        mesh=_sc_mesh(),
        scratch_types=base_scratch + [
            pltpu.VMEM((N,), jnp.float32),
            pltpu.VMEM((N,), jnp.float32),
            pltpu.VMEM((ch, K), jnp.int32),
            pltpu.VMEM((64,), jnp.float32),
        ],
        compiler_params=pltpu.CompilerParams(needs_layout_passes=False),
        interpret=_INTERPRET,
    )
    return f(rips_s, kde, state[0], state[1])


def kernel(x):
    x = jnp.asarray(x, jnp.float32)
    sq = jnp.sum(x * x, axis=1)
    offsum, diag = _k1(x, sq)
    res = (offsum + jnp.exp(-jnp.maximum(diag, 0.0) / SCALE)) / (K * SCALE)
    rank, xaug = _k2(res, x, sq)
    xaug_s = _k3(xaug, rank)
    xs = xaug_s[:, :D]
    sqs = xaug_s[:, D]
    kde = xaug_s[:, D + 1]
    xsT = xs.T
    sqs_col = sqs.reshape(N, 1)
    sqs_row = sqs.reshape(1, N)
    state = (jnp.arange(N, dtype=jnp.int32), jnp.full((N,), -1, jnp.int32))
    for s in range(len(SIZES)):
        rips_s = _k4(s, xs, xsT, sqs_col, sqs_row)
        if s < len(SIZES) - 1:
            state = _k5(s, rips_s, kde, state)
        else:
            misc, topa, topb, topv = _k5(s, rips_s, kde, state)
    s_all, n_app = misc[0], misc[1]
    valid = topv[:DESTNUM] > 0.5
    a = topa[:DESTNUM]
    b = topb[:DESTNUM]
    s_top = jnp.sum(jnp.where(valid, a - b, 0.0))
    weak = (s_all - s_top) / math.sqrt(2.0)
    dest_a, dest_b = topa[0], topb[0]
    dists = jnp.sqrt((a - dest_a) ** 2 + (b - dest_b) ** 2)
    strong = jnp.sum(jnp.where(valid, dists, 0.0))
    return jnp.where(n_app > 0.5, weak + strong, jnp.float32(0.0))
